# baseline (speedup 1.0000x reference)
import functools

import jax
import jax.numpy as jnp
from jax import lax
from jax.experimental import pallas as pl
from jax.experimental.pallas import tpu as pltpu
from jax.experimental.pallas import tpu_sc as plsc

NC = 2
NS = 16
LANES = 16
NW = NC * NS


def _detile(ctab_t, xtab_t, ctail, xtail):
    D, V = ctab_t.shape
    W = 256
    P = W
    n_full = (V // 128) * 128
    n_slabs = n_full // W
    tail = V - n_full
    per_w = (n_slabs + NW - 1) // NW

    mesh = plsc.VectorSubcoreMesh(
        core_axis_name="c", subcore_axis_name="s",
        num_cores=NC, num_subcores=NS)

    assert tail and tail % LANES == 0

    @functools.partial(
        pl.kernel,
        out_type=[jax.ShapeDtypeStruct((V * D,), jnp.float32),
                  jax.ShapeDtypeStruct((V * D,), jnp.float32)],
        mesh=mesh,
        scratch_types=[
            pltpu.VMEM((D * P,), jnp.float32),
            pltpu.VMEM((D * P,), jnp.float32),
            pltpu.VMEM((W * D,), jnp.float32),
            pltpu.VMEM((W * D,), jnp.float32),
            pltpu.SemaphoreType.DMA,
            pltpu.SemaphoreType.DMA,
            pltpu.SemaphoreType.DMA,
            pltpu.SemaphoreType.DMA,
        ],
        compiler_params=pltpu.CompilerParams(
            needs_layout_passes=False, use_tc_tiling_on_sc=True),
    )
    def k(ca_hbm, xa_hbm, ct_hbm, xt_hbm, cl_hbm, xl_hbm,
          in_a, in_b, out_a, out_b, sem_a, sem_b, soa, sob):
        wid = lax.axis_index("s") * NC + lax.axis_index("c")
        iota = lax.iota(jnp.int32, LANES)
        col_p = [(iota + LANES * kk) * W for kk in range(D // LANES)]

        def start_in(tab, in_v, sem, slab):
            c0 = slab * W
            for d in range(D):
                pltpu.async_copy(tab.at[d, pl.ds(c0, W)],
                                 in_v.at[pl.ds(d * P, W)], sem)

        def wait_in(tab, in_v, sem):
            for d in range(D):
                pltpu.make_async_copy(tab.at[d, pl.ds(0, W)],
                                      in_v.at[pl.ds(d * P, W)], sem).wait()

        UNROLL = 8

        def transpose(in_v, out_v):
            def cbody(cc, _):
                c = cc * UNROLL
                vs = []
                for u in range(UNROLL):
                    for kk in range(D // LANES):
                        vs.append(plsc.load_gather(
                            in_v, [col_p[kk] + (c + u)]))
                for u in range(UNROLL):
                    for kk in range(D // LANES):
                        out_v[pl.ds((c + u) * D + LANES * kk, LANES)] = (
                            vs[u * (D // LANES) + kk])
                return _
            lax.fori_loop(0, W // UNROLL, cbody, None)

        def out_copy(out_v, lin, sem, slab):
            pltpu.async_copy(
                out_v, lin.at[pl.ds(slab * W * D, W * D)], sem)

        def wait_out(out_v, lin, sem):
            pltpu.make_async_copy(
                out_v, lin.at[pl.ds(0, W * D)], sem).wait()

        def first_slab(g):
            return jnp.minimum(wid + NW * g, n_slabs - 1)

        start_in(ca_hbm, in_a, sem_a, first_slab(0))

        def body(g, _):
            s0 = first_slab(g)
            live = wid + NW * g < n_slabs
            start_in(xa_hbm, in_b, sem_b, s0)
            wait_in(ca_hbm, in_a, sem_a)

            @pl.when(g > 0)
            def _wo():
                wait_out(out_a, cl_hbm, soa)

            @pl.when(live)
            def _ca():
                transpose(in_a, out_a)
                out_copy(out_a, cl_hbm, soa, s0)

            @pl.when(g + 1 < per_w)
            def _sa():
                start_in(ca_hbm, in_a, sem_a, first_slab(g + 1))
            wait_in(xa_hbm, in_b, sem_b)

            @pl.when(g > 0)
            def _wo2():
                wait_out(out_b, xl_hbm, sob)

            @pl.when(live)
            def _cb():
                transpose(in_b, out_b)
                out_copy(out_b, xl_hbm, sob, s0)
            return _

        lax.fori_loop(0, per_w, body, None)

        @pl.when(wid + NW * (per_w - 1) < n_slabs)
        def _wfinal():
            wait_out(out_a, cl_hbm, soa)
            wait_out(out_b, xl_hbm, sob)

        @pl.when(wid == 0)
        def _tail():
            for t_hbm, lin, in_v, sem in (
                    (ct_hbm, cl_hbm, in_a, sem_a),
                    (xt_hbm, xl_hbm, in_b, sem_b)):
                pltpu.sync_copy(t_hbm, in_v.at[pl.ds(0, tail * D)])
                pltpu.sync_copy(in_v.at[pl.ds(0, tail * D)],
                                lin.at[pl.ds(n_full * D, tail * D)])

    return k(ctab_t, xtab_t, ctail, xtail)


@functools.partial(jax.jit, static_argnames=("interpret",))
def _w2v(center, ctx_t, center_table, context_table, *, interpret=False):
    B = center.shape[0]
    V, D = center_table.shape
    L = ctx_t.shape[0]
    assert D == 64 and B % NW == 0 and V % LANES == 0

    n_full = (V // 128) * 128
    ctab_lin, xtab_lin = _detile(
        center_table.T, context_table.T,
        center_table[n_full:].reshape(-1),
        context_table[n_full:].reshape(-1))
    ctab = ctab_lin.reshape(V, D)
    xtab = xtab_lin.reshape(V, D)

    bpw = B // NW
    C = 32
    assert bpw % (2 * C) == 0
    n_chunks = bpw // C
    rows_per_chunk = C * L

    mesh = plsc.VectorSubcoreMesh(
        core_axis_name="c", subcore_axis_name="s",
        num_cores=NC, num_subcores=NS)

    @functools.partial(
        pl.kernel,
        out_type=jax.ShapeDtypeStruct((L, B), jnp.float32),
        mesh=mesh,
        scratch_types=[
            pltpu.VMEM((bpw,), jnp.int32),
            pltpu.VMEM((L, bpw), jnp.int32),
            pltpu.VMEM((C, D), jnp.float32),
            pltpu.VMEM((C, D), jnp.float32),
            pltpu.VMEM((rows_per_chunk, D), jnp.float32),
            pltpu.VMEM((rows_per_chunk, D), jnp.float32),
            pltpu.VMEM((L, C), jnp.float32),
            pltpu.VMEM((L, C), jnp.float32),
            pltpu.SemaphoreType.DMA,
            pltpu.SemaphoreType.DMA,
            pltpu.SemaphoreType.DMA,
        ],
        compiler_params=pltpu.CompilerParams(
            needs_layout_passes=False, use_tc_tiling_on_sc=False),
        interpret=interpret,
    )
    def k(cen_hbm, ctx_hbm, ctab_hbm, xtab_hbm, out_hbm,
          cen_idx, ctx_idx, cen_a, cen_b, ctx_a, ctx_b, out_a, out_b,
          sem_a, sem_b, sem_o):
        wid = lax.axis_index("s") * NC + lax.axis_index("c")
        base = wid * bpw

        pltpu.sync_copy(cen_hbm.at[pl.ds(base, bpw)], cen_idx)
        pltpu.sync_copy(ctx_hbm.at[:, pl.ds(base, bpw)], ctx_idx)

        def start_gathers(chunk, cen_rows, ctx_rows, sem):
            off = chunk * C
            pltpu.async_copy(
                ctab_hbm.at[cen_idx.at[pl.ds(off, C)]], cen_rows, sem)
            for l in range(L):
                pltpu.async_copy(
                    xtab_hbm.at[ctx_idx.at[l, pl.ds(off, C)]],
                    ctx_rows.at[pl.ds(l * C, C)], sem)

        def wait_gathers(cen_rows, ctx_rows, sem):
            pltpu.make_async_copy(
                ctab_hbm.at[cen_idx.at[pl.ds(0, C)]], cen_rows, sem).wait()
            for l in range(L):
                pltpu.make_async_copy(
                    xtab_hbm.at[ctx_idx.at[l, pl.ds(0, C)]],
                    ctx_rows.at[pl.ds(l * C, C)], sem).wait()

        def compute(chunk, cen_rows, ctx_rows, out_v):
            lanes = lax.iota(jnp.int32, LANES)
            for grp in range(C // LANES):
                def grp_body(i16, res):
                    i = grp * LANES + i16
                    mask = lanes == i16
                    cen = [cen_rows[i, pl.ds(16 * kk, 16)]
                           for kk in range(4)]
                    new_res = []
                    for l in range(L):
                        j = l * C + i
                        s = ctx_rows[j, pl.ds(0, 16)] * cen[0]
                        for kk in range(1, 4):
                            s = s + ctx_rows[j, pl.ds(16 * kk, 16)] * cen[kk]
                        tot = jnp.full((LANES,), jnp.sum(s), jnp.float32)
                        new_res.append(jnp.where(mask, tot, res[l]))
                    return tuple(new_res)

                res = lax.fori_loop(
                    0, LANES, grp_body,
                    tuple(jnp.zeros((LANES,), jnp.float32)
                          for _ in range(L)))
                for l in range(L):
                    out_v[l, pl.ds(grp * LANES, LANES)] = res[l]
            pltpu.async_copy(
                out_v, out_hbm.at[:, pl.ds(base + chunk * C, C)], sem_o)

        def wait_out(out_v, chunk):
            pltpu.make_async_copy(
                out_v, out_hbm.at[:, pl.ds(base + chunk * C, C)],
                sem_o).wait()

        start_gathers(0, cen_a, ctx_a, sem_a)

        def pair_body(g, _):
            c0 = 2 * g
            start_gathers(c0 + 1, cen_b, ctx_b, sem_b)
            wait_gathers(cen_a, ctx_a, sem_a)

            @pl.when(g > 0)
            def _w():
                wait_out(out_a, c0 - 2)
            compute(c0, cen_a, ctx_a, out_a)

            @pl.when(c0 + 2 < n_chunks)
            def _s():
                start_gathers(c0 + 2, cen_a, ctx_a, sem_a)
            wait_gathers(cen_b, ctx_b, sem_b)

            @pl.when(g > 0)
            def _w2():
                wait_out(out_b, c0 - 1)
            compute(c0 + 1, cen_b, ctx_b, out_b)
            return _

        lax.fori_loop(0, n_chunks // 2, pair_body, None)
        wait_out(out_a, n_chunks - 2)
        wait_out(out_b, n_chunks - 1)

    return k(center, ctx_t, ctab, xtab)


def kernel(center, context, center_table, context_table):
    B = center.shape[0]
    L = context.shape[1]
    out_t = _w2v(center, context.T, center_table, context_table)
    return out_t.T


# speedup vs baseline: 1.0888x; 1.0888x over previous
import functools

import jax
import jax.numpy as jnp
from jax import lax
from jax.experimental import pallas as pl
from jax.experimental.pallas import tpu as pltpu
from jax.experimental.pallas import tpu_sc as plsc

NC = 2
NS = 16
LANES = 16
NW = NC * NS


def _detile(ctab_t, xtab_t, ctail, xtail):
    D, V = ctab_t.shape
    W = 256
    P = W
    n_full = (V // 128) * 128
    n_slabs = n_full // W
    tail = V - n_full
    per_w = (n_slabs + NW - 1) // NW

    mesh = plsc.VectorSubcoreMesh(
        core_axis_name="c", subcore_axis_name="s",
        num_cores=NC, num_subcores=NS)

    assert tail and tail % LANES == 0

    @functools.partial(
        pl.kernel,
        out_type=[jax.ShapeDtypeStruct((V * D,), jnp.float32),
                  jax.ShapeDtypeStruct((V * D,), jnp.float32)],
        mesh=mesh,
        scratch_types=[
            pltpu.VMEM((D * P,), jnp.float32),
            pltpu.VMEM((D * P,), jnp.float32),
            pltpu.VMEM((W * D,), jnp.float32),
            pltpu.VMEM((W * D,), jnp.float32),
            pltpu.SemaphoreType.DMA,
            pltpu.SemaphoreType.DMA,
            pltpu.SemaphoreType.DMA,
            pltpu.SemaphoreType.DMA,
        ],
        compiler_params=pltpu.CompilerParams(
            needs_layout_passes=False, use_tc_tiling_on_sc=True),
    )
    def k(ca_hbm, xa_hbm, ct_hbm, xt_hbm, cl_hbm, xl_hbm,
          in_a, in_b, out_a, out_b, sem_a, sem_b, soa, sob):
        wid = lax.axis_index("s") * NC + lax.axis_index("c")
        iota = lax.iota(jnp.int32, LANES)
        col_p = [(iota + LANES * kk) * W for kk in range(D // LANES)]

        def start_in(tab, in_v, sem, slab):
            c0 = slab * W
            for d in range(D):
                pltpu.async_copy(tab.at[d, pl.ds(c0, W)],
                                 in_v.at[pl.ds(d * P, W)], sem)

        def wait_in(tab, in_v, sem):
            for d in range(D):
                pltpu.make_async_copy(tab.at[d, pl.ds(0, W)],
                                      in_v.at[pl.ds(d * P, W)], sem).wait()

        UNROLL = 2

        def transpose(in_v, out_v):
            def cbody(cc, _):
                c = cc * UNROLL
                vs = []
                for u in range(UNROLL):
                    for kk in range(D // LANES):
                        vs.append(plsc.load_gather(
                            in_v, [col_p[kk] + (c + u)]))
                for u in range(UNROLL):
                    for kk in range(D // LANES):
                        out_v[pl.ds((c + u) * D + LANES * kk, LANES)] = (
                            vs[u * (D // LANES) + kk])
                return _
            lax.fori_loop(0, W // UNROLL, cbody, None)

        def out_copy(out_v, lin, sem, slab):
            pltpu.async_copy(
                out_v, lin.at[pl.ds(slab * W * D, W * D)], sem)

        def wait_out(out_v, lin, sem):
            pltpu.make_async_copy(
                out_v, lin.at[pl.ds(0, W * D)], sem).wait()

        def first_slab(g):
            return jnp.minimum(wid + NW * g, n_slabs - 1)

        start_in(ca_hbm, in_a, sem_a, first_slab(0))

        def body(g, _):
            s0 = first_slab(g)
            live = wid + NW * g < n_slabs
            start_in(xa_hbm, in_b, sem_b, s0)
            wait_in(ca_hbm, in_a, sem_a)

            @pl.when(g > 0)
            def _wo():
                wait_out(out_a, cl_hbm, soa)

            @pl.when(live)
            def _ca():
                transpose(in_a, out_a)
                out_copy(out_a, cl_hbm, soa, s0)

            @pl.when(g + 1 < per_w)
            def _sa():
                start_in(ca_hbm, in_a, sem_a, first_slab(g + 1))
            wait_in(xa_hbm, in_b, sem_b)

            @pl.when(g > 0)
            def _wo2():
                wait_out(out_b, xl_hbm, sob)

            @pl.when(live)
            def _cb():
                transpose(in_b, out_b)
                out_copy(out_b, xl_hbm, sob, s0)
            return _

        lax.fori_loop(0, per_w, body, None)

        @pl.when(wid + NW * (per_w - 1) < n_slabs)
        def _wfinal():
            wait_out(out_a, cl_hbm, soa)
            wait_out(out_b, xl_hbm, sob)

        @pl.when(wid == 0)
        def _tail():
            for t_hbm, lin, in_v, sem in (
                    (ct_hbm, cl_hbm, in_a, sem_a),
                    (xt_hbm, xl_hbm, in_b, sem_b)):
                pltpu.sync_copy(t_hbm, in_v.at[pl.ds(0, tail * D)])
                pltpu.sync_copy(in_v.at[pl.ds(0, tail * D)],
                                lin.at[pl.ds(n_full * D, tail * D)])

    return k(ctab_t, xtab_t, ctail, xtail)


@functools.partial(jax.jit, static_argnames=("interpret",))
def _w2v(center, ctx_t, center_table, context_table, *, interpret=False):
    B = center.shape[0]
    V, D = center_table.shape
    L = ctx_t.shape[0]
    assert D == 64 and B % NW == 0 and V % LANES == 0

    n_full = (V // 128) * 128
    ctab_lin, xtab_lin = _detile(
        center_table.T, context_table.T,
        center_table[n_full:].reshape(-1),
        context_table[n_full:].reshape(-1))
    ctab = ctab_lin.reshape(V, D)
    xtab = xtab_lin.reshape(V, D)

    bpw = B // NW
    C = 32
    assert bpw % (2 * C) == 0
    n_chunks = bpw // C
    rows_per_chunk = C * L

    mesh = plsc.VectorSubcoreMesh(
        core_axis_name="c", subcore_axis_name="s",
        num_cores=NC, num_subcores=NS)

    @functools.partial(
        pl.kernel,
        out_type=jax.ShapeDtypeStruct((L, B), jnp.float32),
        mesh=mesh,
        scratch_types=[
            pltpu.VMEM((bpw,), jnp.int32),
            pltpu.VMEM((L, bpw), jnp.int32),
            pltpu.VMEM((C, D), jnp.float32),
            pltpu.VMEM((C, D), jnp.float32),
            pltpu.VMEM((rows_per_chunk, D), jnp.float32),
            pltpu.VMEM((rows_per_chunk, D), jnp.float32),
            pltpu.VMEM((L, C), jnp.float32),
            pltpu.VMEM((L, C), jnp.float32),
            pltpu.SemaphoreType.DMA,
            pltpu.SemaphoreType.DMA,
            pltpu.SemaphoreType.DMA,
        ],
        compiler_params=pltpu.CompilerParams(
            needs_layout_passes=False, use_tc_tiling_on_sc=False),
        interpret=interpret,
    )
    def k(cen_hbm, ctx_hbm, ctab_hbm, xtab_hbm, out_hbm,
          cen_idx, ctx_idx, cen_a, cen_b, ctx_a, ctx_b, out_a, out_b,
          sem_a, sem_b, sem_o):
        wid = lax.axis_index("s") * NC + lax.axis_index("c")
        base = wid * bpw

        pltpu.sync_copy(cen_hbm.at[pl.ds(base, bpw)], cen_idx)
        pltpu.sync_copy(ctx_hbm.at[:, pl.ds(base, bpw)], ctx_idx)

        def start_gathers(chunk, cen_rows, ctx_rows, sem):
            off = chunk * C
            pltpu.async_copy(
                ctab_hbm.at[cen_idx.at[pl.ds(off, C)]], cen_rows, sem)
            for l in range(L):
                pltpu.async_copy(
                    xtab_hbm.at[ctx_idx.at[l, pl.ds(off, C)]],
                    ctx_rows.at[pl.ds(l * C, C)], sem)

        def wait_gathers(cen_rows, ctx_rows, sem):
            pltpu.make_async_copy(
                ctab_hbm.at[cen_idx.at[pl.ds(0, C)]], cen_rows, sem).wait()
            for l in range(L):
                pltpu.make_async_copy(
                    xtab_hbm.at[ctx_idx.at[l, pl.ds(0, C)]],
                    ctx_rows.at[pl.ds(l * C, C)], sem).wait()

        def compute(chunk, cen_rows, ctx_rows, out_v):
            lanes = lax.iota(jnp.int32, LANES)
            for grp in range(C // LANES):
                def grp_body(i16, res):
                    i = grp * LANES + i16
                    mask = lanes == i16
                    cen = [cen_rows[i, pl.ds(16 * kk, 16)]
                           for kk in range(4)]
                    new_res = []
                    for l in range(L):
                        j = l * C + i
                        s = ctx_rows[j, pl.ds(0, 16)] * cen[0]
                        for kk in range(1, 4):
                            s = s + ctx_rows[j, pl.ds(16 * kk, 16)] * cen[kk]
                        tot = jnp.full((LANES,), jnp.sum(s), jnp.float32)
                        new_res.append(jnp.where(mask, tot, res[l]))
                    return tuple(new_res)

                res = lax.fori_loop(
                    0, LANES, grp_body,
                    tuple(jnp.zeros((LANES,), jnp.float32)
                          for _ in range(L)))
                for l in range(L):
                    out_v[l, pl.ds(grp * LANES, LANES)] = res[l]
            pltpu.async_copy(
                out_v, out_hbm.at[:, pl.ds(base + chunk * C, C)], sem_o)

        def wait_out(out_v, chunk):
            pltpu.make_async_copy(
                out_v, out_hbm.at[:, pl.ds(base + chunk * C, C)],
                sem_o).wait()

        start_gathers(0, cen_a, ctx_a, sem_a)

        def pair_body(g, _):
            c0 = 2 * g
            start_gathers(c0 + 1, cen_b, ctx_b, sem_b)
            wait_gathers(cen_a, ctx_a, sem_a)

            @pl.when(g > 0)
            def _w():
                wait_out(out_a, c0 - 2)
            compute(c0, cen_a, ctx_a, out_a)

            @pl.when(c0 + 2 < n_chunks)
            def _s():
                start_gathers(c0 + 2, cen_a, ctx_a, sem_a)
            wait_gathers(cen_b, ctx_b, sem_b)

            @pl.when(g > 0)
            def _w2():
                wait_out(out_b, c0 - 1)
            compute(c0 + 1, cen_b, ctx_b, out_b)
            return _

        lax.fori_loop(0, n_chunks // 2, pair_body, None)
        wait_out(out_a, n_chunks - 2)
        wait_out(out_b, n_chunks - 1)

    return k(center, ctx_t, ctab, xtab)


def kernel(center, context, center_table, context_table):
    B = center.shape[0]
    L = context.shape[1]
    out_t = _w2v(center, context.T, center_table, context_table)
    return out_t.T


# speedup vs baseline: 2.7028x; 2.4824x over previous
import functools

import jax
import jax.numpy as jnp
from jax import lax
from jax.experimental import pallas as pl
from jax.experimental.pallas import tpu as pltpu
from jax.experimental.pallas import tpu_sc as plsc

NC = 2
NS = 16
LANES = 16
NW = NC * NS


@functools.partial(jax.jit, static_argnames=("interpret",))
def _w2v(cen_embed, ctx_t, context_table, *, interpret=False):
    B, D = cen_embed.shape
    V, _ = context_table.shape
    L = ctx_t.shape[0]
    assert D == 64 and B % NW == 0
    bpw = B // NW
    C = 32
    assert bpw % (2 * C) == 0
    n_chunks = bpw // C
    rows_per_chunk = C * L

    mesh = plsc.VectorSubcoreMesh(
        core_axis_name="c", subcore_axis_name="s",
        num_cores=NC, num_subcores=NS)

    @functools.partial(
        pl.kernel,
        out_type=jax.ShapeDtypeStruct((L, B), jnp.float32),
        mesh=mesh,
        scratch_types=[
            pltpu.VMEM((L, bpw), jnp.int32),
            pltpu.VMEM((C, D), jnp.float32),
            pltpu.VMEM((C, D), jnp.float32),
            pltpu.VMEM((rows_per_chunk, D), jnp.float32),
            pltpu.VMEM((rows_per_chunk, D), jnp.float32),
            pltpu.VMEM((L, C), jnp.float32),
            pltpu.VMEM((L, C), jnp.float32),
            pltpu.SemaphoreType.DMA,
            pltpu.SemaphoreType.DMA,
            pltpu.SemaphoreType.DMA,
        ],
        compiler_params=pltpu.CompilerParams(
            needs_layout_passes=False, use_tc_tiling_on_sc=False),
        interpret=interpret,
    )
    def k(cen_hbm, ctx_hbm, xtab_hbm, out_hbm,
          ctx_idx, cen_a, cen_b, ctx_a, ctx_b, out_a, out_b,
          sem_a, sem_b, sem_o):
        wid = lax.axis_index("s") * NC + lax.axis_index("c")
        base = wid * bpw

        pltpu.sync_copy(ctx_hbm.at[:, pl.ds(base, bpw)], ctx_idx)

        def start_gathers(chunk, cen_rows, ctx_rows, sem):
            off = chunk * C
            pltpu.async_copy(
                cen_hbm.at[pl.ds(base + off, C), :], cen_rows, sem)
            for l in range(L):
                pltpu.async_copy(
                    xtab_hbm.at[ctx_idx.at[l, pl.ds(off, C)]],
                    ctx_rows.at[pl.ds(l * C, C)], sem)

        def wait_gathers(cen_rows, ctx_rows, sem):
            pltpu.make_async_copy(
                cen_hbm.at[pl.ds(0, C), :], cen_rows, sem).wait()
            for l in range(L):
                pltpu.make_async_copy(
                    xtab_hbm.at[ctx_idx.at[l, pl.ds(0, C)]],
                    ctx_rows.at[pl.ds(l * C, C)], sem).wait()

        def compute(chunk, cen_rows, ctx_rows, out_v):
            lanes = lax.iota(jnp.int32, LANES)
            for grp in range(C // LANES):
                def grp_body(i16, res):
                    i = grp * LANES + i16
                    mask = lanes == i16
                    cen = [cen_rows[i, pl.ds(16 * kk, 16)]
                           for kk in range(4)]
                    new_res = []
                    for l in range(L):
                        j = l * C + i
                        s = ctx_rows[j, pl.ds(0, 16)] * cen[0]
                        for kk in range(1, 4):
                            s = s + ctx_rows[j, pl.ds(16 * kk, 16)] * cen[kk]
                        tot = jnp.full((LANES,), jnp.sum(s), jnp.float32)
                        new_res.append(jnp.where(mask, tot, res[l]))
                    return tuple(new_res)

                res = lax.fori_loop(
                    0, LANES, grp_body,
                    tuple(jnp.zeros((LANES,), jnp.float32)
                          for _ in range(L)))
                for l in range(L):
                    out_v[l, pl.ds(grp * LANES, LANES)] = res[l]
            pltpu.async_copy(
                out_v, out_hbm.at[:, pl.ds(base + chunk * C, C)], sem_o)

        def wait_out(out_v, chunk):
            pltpu.make_async_copy(
                out_v, out_hbm.at[:, pl.ds(base + chunk * C, C)],
                sem_o).wait()

        start_gathers(0, cen_a, ctx_a, sem_a)

        def pair_body(g, _):
            c0 = 2 * g
            start_gathers(c0 + 1, cen_b, ctx_b, sem_b)
            wait_gathers(cen_a, ctx_a, sem_a)

            @pl.when(g > 0)
            def _w():
                wait_out(out_a, c0 - 2)
            compute(c0, cen_a, ctx_a, out_a)

            @pl.when(c0 + 2 < n_chunks)
            def _s():
                start_gathers(c0 + 2, cen_a, ctx_a, sem_a)
            wait_gathers(cen_b, ctx_b, sem_b)

            @pl.when(g > 0)
            def _w2():
                wait_out(out_b, c0 - 1)
            compute(c0 + 1, cen_b, ctx_b, out_b)
            return _

        lax.fori_loop(0, n_chunks // 2, pair_body, None)
        wait_out(out_a, n_chunks - 2)
        wait_out(out_b, n_chunks - 1)

    return k(cen_embed, ctx_t, context_table)


def kernel(center, context, center_table, context_table):
    B = center.shape[0]
    L = context.shape[1]
    cen_embed = jnp.take(center_table, center, axis=0)
    out_t = _w2v(cen_embed, context.T, context_table)
    return out_t.T
